# baseline (speedup 1.0000x reference)
import functools

import jax
import jax.numpy as jnp
from jax import lax
from jax.experimental import pallas as pl
from jax.experimental.pallas import tpu as pltpu
from jax.experimental.pallas import tpu_sc as plsc

_N = 100000
_D = 128
_NW = 32
_B = 80
_CHUNKS = 80
_CH_A = 50
_CH_B = 30
_NPAD = 16 * _B * _CHUNKS


_BM = 1024


def _mm_body(x_ref, wc_ref, b_ref, o0, o1, o2, o3, o4):
    y = jnp.dot(x_ref[...], wc_ref[...], preferred_element_type=jnp.float32)
    o0[...] = y[:, 0 * _D:1 * _D] + b_ref[...]
    o1[...] = y[:, 1 * _D:2 * _D]
    o2[...] = y[:, 2 * _D:3 * _D]
    o3[...] = y[:, 3 * _D:4 * _D]
    o4[...] = y[:, 4 * _D:5 * _D]


def _tc_tables(xp, wc, b):
    grid = _NPAD // _BM
    out_sd = jax.ShapeDtypeStruct((_NPAD, _D), jnp.float32)
    obs = pl.BlockSpec((_BM, _D), lambda i: (i, 0))
    return pl.pallas_call(
        _mm_body,
        grid=(grid,),
        in_specs=[
            pl.BlockSpec((_BM, _D), lambda i: (i, 0)),
            pl.BlockSpec((_D, 5 * _D), lambda i: (0, 0)),
            pl.BlockSpec((1, _D), lambda i: (0, 0)),
        ],
        out_specs=[obs, obs, obs, obs, obs],
        out_shape=[out_sd, out_sd, out_sd, out_sd, out_sd],
    )(xp, wc, b)


def _sc_body(y0_hbm, y1_hbm, y2_hbm, y3_hbm, y4_hbm,
             i0_hbm, i1_hbm, i2_hbm, i3_hbm,
             out_hbm, *scr):
    sets = []
    for sidx in range(2):
        o = sidx * 9
        sets.append(dict(
            xv=scr[o:o + 4], acc=scr[o + 4], gv=scr[o + 5:o + 9],
            sems=scr[18 + sidx * 5:18 + sidx * 5 + 5],
        ))
    ih = (i0_hbm, i1_hbm, i2_hbm, i3_hbm)
    tbl = (y1_hbm, y2_hbm, y3_hbm, y4_hbm)
    cc = lax.axis_index("c")
    ss = lax.axis_index("s")
    nch = jnp.where(cc == 0, _CH_A, _CH_B)
    base0 = jnp.where(cc == 0, ss * _CH_A, 16 * _CH_A + ss * _CH_B) * _B

    def issue(ci, st):
        base = base0 + ci * _B
        for k in range(4):
            pltpu.sync_copy(ih[k].at[pl.ds(base, _B)], st["xv"][k])
        for k in range(4):
            pltpu.async_copy(tbl[k].at[st["xv"][k]], st["gv"][k],
                             st["sems"][k])
        pltpu.async_copy(y0_hbm.at[pl.ds(base, _B)], st["acc"],
                         st["sems"][4])

    def finish(ci, st):
        base = base0 + ci * _B
        acc_v = st["acc"]
        g0_v, g1_v, g2_v, g3_v = st["gv"]
        for k in range(4):
            pltpu.make_async_copy(tbl[k].at[pl.ds(0, _B)], st["gv"][k],
                                  st["sems"][k]).wait()
        pltpu.make_async_copy(y0_hbm.at[pl.ds(0, _B)], acc_v,
                              st["sems"][4]).wait()

        def row_body(r, rcarry):
            for c in range(_D // 16):
                s = pl.ds(c * 16, 16)
                v = (acc_v[r, s] + g0_v[r, s] + g1_v[r, s]
                     + g2_v[r, s] + g3_v[r, s])
                acc_v[r, s] = jnp.maximum(v, 0.0)
            return rcarry

        lax.fori_loop(0, _B, row_body, 0)
        pltpu.sync_copy(acc_v, out_hbm.at[pl.ds(base, _B)])

    issue(0, sets[0])

    def pair_body(i, carry):
        c0 = 2 * i
        issue(c0 + 1, sets[1])
        finish(c0, sets[0])

        @pl.when(c0 + 2 < nch)
        def _():
            issue(c0 + 2, sets[0])

        finish(c0 + 1, sets[1])
        return carry

    lax.fori_loop(0, nch // 2, pair_body, 0)


@functools.cache
def _sc_gather_sum_kernel():
    scratch = []
    for _ in range(2):
        scratch += [pltpu.VMEM((_B,), jnp.int32) for _ in range(4)]
        scratch += [pltpu.VMEM((_B, _D), jnp.float32) for _ in range(5)]
    scratch += [pltpu.SemaphoreType.DMA for _ in range(10)]
    return pl.kernel(
        _sc_body,
        mesh=plsc.VectorSubcoreMesh(core_axis_name="c", subcore_axis_name="s"),
        out_type=jax.ShapeDtypeStruct((_NPAD, _D), jnp.float32),
        scratch_types=scratch,
    )


def _sc_gather_sum(*args):
    return _sc_gather_sum_kernel()(*args)


def kernel(x, neighbors, W0, b0, W1, b1):
    xp = jnp.pad(x, ((0, _NPAD - _N), (0, 0)))
    nb = jnp.pad(neighbors.astype(jnp.int32), ((0, _NPAD - _N), (0, 0)))
    i0 = nb[:, 0]
    i1 = nb[:, 1]
    i2 = nb[:, 2]
    i3 = nb[:, 3]

    def wcat(W):
        return W.reshape(5, _D, _D).transpose(1, 0, 2).reshape(_D, 5 * _D)

    y = _tc_tables(xp, wcat(W0), b0.reshape(1, _D))
    h1 = _sc_gather_sum(y[0], y[1], y[2], y[3], y[4], i0, i1, i2, i3)
    y = _tc_tables(h1, wcat(W1), b1.reshape(1, _D))
    h2 = _sc_gather_sum(y[0], y[1], y[2], y[3], y[4], i0, i1, i2, i3)
    return h2[:_N]


# speedup vs baseline: 1.0147x; 1.0147x over previous
import functools

import jax
import jax.numpy as jnp
from jax import lax
from jax.experimental import pallas as pl
from jax.experimental.pallas import tpu as pltpu
from jax.experimental.pallas import tpu_sc as plsc

_N = 100000
_D = 128
_NW = 32
_B = 80
_CHUNKS = 80
_CH_A = 52
_CH_B = 28
_NPAD = 16 * _B * _CHUNKS


_BM = 1024


def _mm_body(x_ref, wc_ref, b_ref, o0, o1, o2, o3, o4):
    y = jnp.dot(x_ref[...], wc_ref[...], preferred_element_type=jnp.float32)
    o0[...] = y[:, 0 * _D:1 * _D] + b_ref[...]
    o1[...] = y[:, 1 * _D:2 * _D]
    o2[...] = y[:, 2 * _D:3 * _D]
    o3[...] = y[:, 3 * _D:4 * _D]
    o4[...] = y[:, 4 * _D:5 * _D]


def _tc_tables(xp, wc, b):
    grid = _NPAD // _BM
    out_sd = jax.ShapeDtypeStruct((_NPAD, _D), jnp.float32)
    obs = pl.BlockSpec((_BM, _D), lambda i: (i, 0))
    return pl.pallas_call(
        _mm_body,
        grid=(grid,),
        in_specs=[
            pl.BlockSpec((_BM, _D), lambda i: (i, 0)),
            pl.BlockSpec((_D, 5 * _D), lambda i: (0, 0)),
            pl.BlockSpec((1, _D), lambda i: (0, 0)),
        ],
        out_specs=[obs, obs, obs, obs, obs],
        out_shape=[out_sd, out_sd, out_sd, out_sd, out_sd],
    )(xp, wc, b)


def _sc_body(y0_hbm, y1_hbm, y2_hbm, y3_hbm, y4_hbm,
             i0_hbm, i1_hbm, i2_hbm, i3_hbm,
             out_hbm, *scr):
    sets = []
    for sidx in range(2):
        o = sidx * 9
        sets.append(dict(
            xv=scr[o:o + 4], acc=scr[o + 4], gv=scr[o + 5:o + 9],
            sems=scr[18 + sidx * 5:18 + sidx * 5 + 5],
        ))
    ih = (i0_hbm, i1_hbm, i2_hbm, i3_hbm)
    tbl = (y1_hbm, y2_hbm, y3_hbm, y4_hbm)
    cc = lax.axis_index("c")
    ss = lax.axis_index("s")
    nch = jnp.where(cc == 0, _CH_A, _CH_B)
    base0 = jnp.where(cc == 0, ss * _CH_A, 16 * _CH_A + ss * _CH_B) * _B

    def issue(ci, st):
        base = base0 + ci * _B
        for k in range(4):
            pltpu.sync_copy(ih[k].at[pl.ds(base, _B)], st["xv"][k])
        for k in range(4):
            pltpu.async_copy(tbl[k].at[st["xv"][k]], st["gv"][k],
                             st["sems"][k])
        pltpu.async_copy(y0_hbm.at[pl.ds(base, _B)], st["acc"],
                         st["sems"][4])

    def finish(ci, st):
        base = base0 + ci * _B
        acc_v = st["acc"]
        g0_v, g1_v, g2_v, g3_v = st["gv"]
        for k in range(4):
            pltpu.make_async_copy(tbl[k].at[pl.ds(0, _B)], st["gv"][k],
                                  st["sems"][k]).wait()
        pltpu.make_async_copy(y0_hbm.at[pl.ds(0, _B)], acc_v,
                              st["sems"][4]).wait()

        def row_body(r, rcarry):
            for c in range(_D // 16):
                s = pl.ds(c * 16, 16)
                v = (acc_v[r, s] + g0_v[r, s] + g1_v[r, s]
                     + g2_v[r, s] + g3_v[r, s])
                acc_v[r, s] = jnp.maximum(v, 0.0)
            return rcarry

        lax.fori_loop(0, _B, row_body, 0)
        pltpu.sync_copy(acc_v, out_hbm.at[pl.ds(base, _B)])

    issue(0, sets[0])

    def pair_body(i, carry):
        c0 = 2 * i
        issue(c0 + 1, sets[1])
        finish(c0, sets[0])

        @pl.when(c0 + 2 < nch)
        def _():
            issue(c0 + 2, sets[0])

        finish(c0 + 1, sets[1])
        return carry

    lax.fori_loop(0, nch // 2, pair_body, 0)


@functools.cache
def _sc_gather_sum_kernel():
    scratch = []
    for _ in range(2):
        scratch += [pltpu.VMEM((_B,), jnp.int32) for _ in range(4)]
        scratch += [pltpu.VMEM((_B, _D), jnp.float32) for _ in range(5)]
    scratch += [pltpu.SemaphoreType.DMA for _ in range(10)]
    return pl.kernel(
        _sc_body,
        mesh=plsc.VectorSubcoreMesh(core_axis_name="c", subcore_axis_name="s"),
        out_type=jax.ShapeDtypeStruct((_NPAD, _D), jnp.float32),
        scratch_types=scratch,
    )


def _sc_gather_sum(*args):
    return _sc_gather_sum_kernel()(*args)


def kernel(x, neighbors, W0, b0, W1, b1):
    xp = jnp.pad(x, ((0, _NPAD - _N), (0, 0)))
    nb = jnp.pad(neighbors.astype(jnp.int32), ((0, _NPAD - _N), (0, 0)))
    i0 = nb[:, 0]
    i1 = nb[:, 1]
    i2 = nb[:, 2]
    i3 = nb[:, 3]

    def wcat(W):
        return W.reshape(5, _D, _D).transpose(1, 0, 2).reshape(_D, 5 * _D)

    y = _tc_tables(xp, wcat(W0), b0.reshape(1, _D))
    h1 = _sc_gather_sum(y[0], y[1], y[2], y[3], y[4], i0, i1, i2, i3)
    y = _tc_tables(h1, wcat(W1), b1.reshape(1, _D))
    h2 = _sc_gather_sum(y[0], y[1], y[2], y[3], y[4], i0, i1, i2, i3)
    return h2[:_N]
